# baseline (speedup 1.0000x reference)
import functools

import jax
import jax.numpy as jnp
from jax import lax
from jax.experimental import pallas as pl
from jax.experimental.pallas import tpu as pltpu
from jax.experimental.pallas import tpu_sc as plsc

NC = 2
NS = 16
NW = NC * NS

CT = 4
D = 8192
N_TOKENS = 8192
N_PER_W = N_TOKENS // NW
N_CHUNKS = N_PER_W // CT


def _sc_body(idx_hbm, table_hbm, out_hbm, idx_v, widx2, xrow, r0, r1, r2,
             si0, si1, si2, so0, so1, so2):
  tbl32 = table_hbm.bitcast(jnp.int32)
  out32 = out_hbm.bitcast(jnp.int32)
  wid = lax.axis_index("s") * NC + lax.axis_index("c")
  base = wid * N_PER_W
  bufs = (r0, r1, r2)
  isems = (si0, si1, si2)
  osems = (so0, so1, so2)

  pltpu.sync_copy(idx_hbm.at[pl.ds(base, N_PER_W)], idx_v.at[pl.ds(0, N_PER_W)])
  lane = lax.iota(jnp.int32, 16)

  def widx_body(g, _):
    pos = g * 16 + lane
    v = idx_v[pl.ds(g * 16, 16)] >> 1
    plsc.store_scatter(widx2, [pos >> 2, pos & 3], v)
    return 0

  lax.fori_loop(0, N_PER_W // 16, widx_body, 0, unroll=4)

  chunk_vecs = D // 16
  inv_d = jnp.float32(1.0 / D)
  inv_dm1 = jnp.float32(1.0 / (D - 1))
  mask = jnp.int32(-65536)

  def start_in(c, b, s):
    pltpu.async_copy(tbl32.at[widx2.at[c]], b, s)

  def wait_in(b, s):
    pltpu.make_async_copy(tbl32.at[widx2.at[0]], b, s).wait()

  def start_out(c, b, s):
    pltpu.async_copy(b, out32.at[pl.ds(base + c * CT, CT)], s)

  def wait_out(c, b, s):
    pltpu.make_async_copy(b, out32.at[pl.ds(base + c * CT, CT)], s).wait()

  def compute(c, rows_v):
    def half_body(t, extract):
      def stats_body(i, carry):
        acc = list(carry)
        for u in range(4):
          w = rows_v[t, pl.ds((i * 4 + u) * 16, 16)]
          x = extract(w)
          xrow[pl.ds((i * 4 + u) * 16, 16)] = x
          acc[2 * u] = acc[2 * u] + x
          acc[2 * u + 1] = acc[2 * u + 1] + x * x
        return tuple(acc)

      z = jnp.zeros((16,), jnp.float32)
      acc = lax.fori_loop(0, chunk_vecs // 4, stats_body, (z,) * 8, unroll=4)
      tot = jnp.sum(acc[0] + acc[2] + acc[4] + acc[6])
      ss = jnp.sum(acc[1] + acc[3] + acc[5] + acc[7])
      mean = tot * inv_d
      var = (ss - tot * mean) * inv_dm1

      vv = jnp.full((16,), var, jnp.float32)
      iy = jnp.int32(0x5F3759DF) - (plsc.bitcast(vv, jnp.int32) >> 1)
      y = plsc.bitcast(iy, jnp.float32)
      half_v = vv * 0.5
      y = y * (1.5 - half_v * y * y)
      y = y * (1.5 - half_v * y * y)
      a = y - 1e-6 * (y * y)
      b = jnp.full((16,), -mean, jnp.float32) * a

      def norm_body(i, _):
        x = xrow[pl.ds(i * 16, 16)]
        rows_v[t, pl.ds(i * 16, 16)] = plsc.bitcast(x * a + b, jnp.int32)
        return 0

      lax.fori_loop(0, chunk_vecs, norm_body, 0, unroll=8)

    def token_body(t):
      par = idx_v[pl.ds(c * CT + t, 16)][0] & 1

      @pl.when(par == 0)
      def _():
        half_body(t, lambda w: plsc.bitcast(w << 16, jnp.float32))

      @pl.when(par != 0)
      def _():
        half_body(t, lambda w: plsc.bitcast(w & mask, jnp.float32))

    for t in range(CT):
      token_body(t)

  start_in(0, r0, si0)
  start_in(1, r1, si1)

  def loop_body(k, _):
    for j in range(3):
      c = 3 * k + j
      b, si, so = bufs[j], isems[j], osems[j]
      nj = (j + 2) % 3
      nb, nsi, nso = bufs[nj], isems[nj], osems[nj]
      wait_in(b, si)
      compute(c, b)
      start_out(c, b, so)

      @pl.when(c + 2 < N_CHUNKS)
      def _():
        @pl.when(c >= 1)
        def _():
          wait_out(c - 1, nb, nso)

        start_in(c + 2, nb, nsi)

    return 0

  lax.fori_loop(0, (N_CHUNKS - 1) // 3, loop_body, 0)

  c_last = N_CHUNKS - 1
  wait_in(r0, si0)
  compute(c_last, r0)
  start_out(c_last, r0, so0)
  wait_out(c_last - 2, r1, so1)
  wait_out(c_last - 1, r2, so2)
  wait_out(c_last, r0, so0)


@jax.jit
def kernel(token_ids, PF_table):
  ids = token_ids.reshape(-1)
  mesh = plsc.VectorSubcoreMesh(core_axis_name="c", subcore_axis_name="s")
  fn = functools.partial(
      pl.kernel,
      out_type=jax.ShapeDtypeStruct((N_TOKENS, D), jnp.float32),
      mesh=mesh,
      compiler_params=pltpu.CompilerParams(needs_layout_passes=False),
      scratch_types=[
          pltpu.VMEM((N_PER_W + 16,), jnp.int32),
          pltpu.VMEM((N_CHUNKS, CT), jnp.int32),
          pltpu.VMEM((D,), jnp.float32),
          pltpu.VMEM((CT, D), jnp.int32),
          pltpu.VMEM((CT, D), jnp.int32),
          pltpu.VMEM((CT, D), jnp.int32),
          pltpu.SemaphoreType.DMA,
          pltpu.SemaphoreType.DMA,
          pltpu.SemaphoreType.DMA,
          pltpu.SemaphoreType.DMA,
          pltpu.SemaphoreType.DMA,
          pltpu.SemaphoreType.DMA,
      ],
  )(_sc_body)
  out = fn(ids, PF_table)
  return out.reshape(*token_ids.shape, D)


# speedup vs baseline: 4.6394x; 4.6394x over previous
import functools

import jax
import jax.numpy as jnp
from jax import lax
from jax.experimental import pallas as pl
from jax.experimental.pallas import tpu as pltpu
from jax.experimental.pallas import tpu_sc as plsc

NC = 2
NS = 16
NW = NC * NS

CT = 4
D = 8192
VOCAB = 8192
N_TOKENS = 8192
N_PER_W = N_TOKENS // NW
N_CHUNKS = N_PER_W // CT
TC_ROWS = 128


def _tc_stats_body(tbl_ref, a_ref, b_ref):
  x = tbl_ref[...].astype(jnp.float32)
  tot = jnp.sum(x, axis=1)
  ss = jnp.sum(x * x, axis=1)
  mean = tot * (1.0 / D)
  var = (ss - tot * mean) * (1.0 / (D - 1))
  a = 1.0 / (jnp.sqrt(var) + 1e-6)
  a_ref[...] = a
  b_ref[...] = -mean * a


def _sc_body(idx_hbm, table_hbm, a_hbm, b_hbm, out_hbm,
             idx_v, widx2, a_v, b_v, r0, r1, r2,
             si0, si1, si2, so0, so1, so2):
  tbl32 = table_hbm.bitcast(jnp.int32)
  out32 = out_hbm.bitcast(jnp.int32)
  wid = lax.axis_index("s") * NC + lax.axis_index("c")
  base = wid * N_PER_W
  bufs = (r0, r1, r2)
  isems = (si0, si1, si2)
  osems = (so0, so1, so2)

  pltpu.sync_copy(idx_hbm.at[pl.ds(base, N_PER_W)], idx_v)
  pltpu.sync_copy(a_hbm, a_v)
  pltpu.sync_copy(b_hbm, b_v)
  lane = lax.iota(jnp.int32, 16)

  def widx_body(g, _):
    pos = g * 16 + lane
    v = idx_v[pl.ds(g * 16, 16)] >> 1
    plsc.store_scatter(widx2, [pos >> 2, pos & 3], v)
    return 0

  lax.fori_loop(0, N_PER_W // 16, widx_body, 0, unroll=4)

  chunk_vecs = D // 16
  mask = jnp.int32(-65536)

  def start_in(c, b, s):
    pltpu.async_copy(tbl32.at[widx2.at[c]], b, s)

  def wait_in(b, s):
    pltpu.make_async_copy(tbl32.at[widx2.at[0]], b, s).wait()

  def start_out(c, b, s):
    pltpu.async_copy(b, out32.at[pl.ds(base + c * CT, CT)], s)

  def wait_out(c, b, s):
    pltpu.make_async_copy(b, out32.at[pl.ds(base + c * CT, CT)], s).wait()

  def compute(c, rows_v):
    def half_body(t, a, b, extract):
      def norm_body(i, _):
        w = rows_v[t, pl.ds(i * 16, 16)]
        rows_v[t, pl.ds(i * 16, 16)] = plsc.bitcast(
            extract(w) * a + b, jnp.int32
        )
        return 0

      lax.fori_loop(0, chunk_vecs, norm_body, 0, unroll=8)

    def token_body(t):
      tid = plsc.load_gather(idx_v, [jnp.full((16,), c * CT + t, jnp.int32)])
      a = plsc.load_gather(a_v, [tid])
      b = plsc.load_gather(b_v, [tid])
      par = jnp.sum(tid & 1)

      @pl.when(par == 0)
      def _():
        half_body(t, a, b, lambda w: plsc.bitcast(w << 16, jnp.float32))

      @pl.when(par != 0)
      def _():
        half_body(t, a, b, lambda w: plsc.bitcast(w & mask, jnp.float32))

    for t in range(CT):
      token_body(t)

  start_in(0, r0, si0)
  start_in(1, r1, si1)

  def loop_body(k, _):
    for j in range(3):
      c = 3 * k + j
      b, si, so = bufs[j], isems[j], osems[j]
      nj = (j + 2) % 3
      nb, nsi, nso = bufs[nj], isems[nj], osems[nj]
      wait_in(b, si)
      compute(c, b)
      start_out(c, b, so)

      @pl.when(c + 2 < N_CHUNKS)
      def _():
        @pl.when(c >= 1)
        def _():
          wait_out(c - 1, nb, nso)

        start_in(c + 2, nb, nsi)

    return 0

  lax.fori_loop(0, (N_CHUNKS - 1) // 3, loop_body, 0)

  c_last = N_CHUNKS - 1
  wait_in(r0, si0)
  compute(c_last, r0)
  start_out(c_last, r0, so0)
  wait_out(c_last - 2, r1, so1)
  wait_out(c_last - 1, r2, so2)
  wait_out(c_last, r0, so0)


@jax.jit
def kernel(token_ids, PF_table):
  ids = token_ids.reshape(-1)

  a_vec, b_vec = pl.pallas_call(
      _tc_stats_body,
      grid=(VOCAB // TC_ROWS,),
      in_specs=[pl.BlockSpec((TC_ROWS, D), lambda i: (i, 0))],
      out_specs=[
          pl.BlockSpec((TC_ROWS,), lambda i: (i,)),
          pl.BlockSpec((TC_ROWS,), lambda i: (i,)),
      ],
      out_shape=[
          jax.ShapeDtypeStruct((VOCAB,), jnp.float32),
          jax.ShapeDtypeStruct((VOCAB,), jnp.float32),
      ],
  )(PF_table)

  mesh = plsc.VectorSubcoreMesh(core_axis_name="c", subcore_axis_name="s")
  fn = functools.partial(
      pl.kernel,
      out_type=jax.ShapeDtypeStruct((N_TOKENS, D), jnp.float32),
      mesh=mesh,
      compiler_params=pltpu.CompilerParams(needs_layout_passes=False),
      scratch_types=[
          pltpu.VMEM((N_PER_W,), jnp.int32),
          pltpu.VMEM((N_CHUNKS, CT), jnp.int32),
          pltpu.VMEM((VOCAB,), jnp.float32),
          pltpu.VMEM((VOCAB,), jnp.float32),
          pltpu.VMEM((CT, D), jnp.int32),
          pltpu.VMEM((CT, D), jnp.int32),
          pltpu.VMEM((CT, D), jnp.int32),
          pltpu.SemaphoreType.DMA,
          pltpu.SemaphoreType.DMA,
          pltpu.SemaphoreType.DMA,
          pltpu.SemaphoreType.DMA,
          pltpu.SemaphoreType.DMA,
          pltpu.SemaphoreType.DMA,
      ],
  )(_sc_body)
  out = fn(ids, PF_table, a_vec, b_vec)
  return out.reshape(*token_ids.shape, D)


# speedup vs baseline: 4.9820x; 1.0739x over previous
import functools

import jax
import jax.numpy as jnp
from jax import lax
from jax.experimental import pallas as pl
from jax.experimental.pallas import tpu as pltpu
from jax.experimental.pallas import tpu_sc as plsc

NC = 2
NS = 16
NW = NC * NS

CT = 4
D = 8192
N_TOKENS = 8192
N_PER_W = N_TOKENS // NW
N_CHUNKS = N_PER_W // CT


def _sc_body(idx_hbm, table_hbm, out_hbm, idx_v, widx2, r0, r1, r2,
             si0, si1, si2, so0, so1, so2):
  tbl32 = table_hbm.bitcast(jnp.int32)
  out32 = out_hbm.bitcast(jnp.int32)
  wid = lax.axis_index("s") * NC + lax.axis_index("c")
  base = wid * N_PER_W
  bufs = (r0, r1, r2)
  isems = (si0, si1, si2)
  osems = (so0, so1, so2)

  pltpu.sync_copy(idx_hbm.at[pl.ds(base, N_PER_W)], idx_v)
  lane = lax.iota(jnp.int32, 16)

  def widx_body(g, _):
    pos = g * 16 + lane
    v = idx_v[pl.ds(g * 16, 16)] >> 1
    plsc.store_scatter(widx2, [pos >> 2, pos & 3], v)
    return 0

  lax.fori_loop(0, N_PER_W // 16, widx_body, 0, unroll=4)

  chunk_vecs = D // 16
  inv_d = jnp.float32(1.0 / D)
  inv_dm1 = jnp.float32(1.0 / (D - 1))
  mask = jnp.int32(-65536)

  def start_in(c, b, s):
    pltpu.async_copy(tbl32.at[widx2.at[c]], b, s)

  def wait_in(b, s):
    pltpu.make_async_copy(tbl32.at[widx2.at[0]], b, s).wait()

  def start_out(c, b, s):
    pltpu.async_copy(b, out32.at[pl.ds(base + c * CT, CT)], s)

  def wait_out(c, b, s):
    pltpu.make_async_copy(b, out32.at[pl.ds(base + c * CT, CT)], s).wait()

  def compute(c, rows_v):
    def half_body(t, extract):
      def stats_body(i, carry):
        acc = list(carry)
        for u in range(4):
          w = rows_v[t, pl.ds((i * 4 + u) * 16, 16)]
          x = extract(w)
          acc[2 * u] = acc[2 * u] + x
          acc[2 * u + 1] = acc[2 * u + 1] + x * x
        return tuple(acc)

      z = jnp.zeros((16,), jnp.float32)
      acc = lax.fori_loop(0, chunk_vecs // 4, stats_body, (z,) * 8, unroll=4)
      tot = jnp.sum(acc[0] + acc[2] + acc[4] + acc[6])
      ss = jnp.sum(acc[1] + acc[3] + acc[5] + acc[7])
      mean = tot * inv_d
      var = (ss - tot * mean) * inv_dm1

      vv = jnp.full((16,), var, jnp.float32)
      iy = jnp.int32(0x5F3759DF) - (plsc.bitcast(vv, jnp.int32) >> 1)
      y = plsc.bitcast(iy, jnp.float32)
      half_v = vv * 0.5
      y = y * (1.5 - half_v * y * y)
      y = y * (1.5 - half_v * y * y)
      a = y - 1e-6 * (y * y)
      b = jnp.full((16,), -mean, jnp.float32) * a

      def norm_body(i, _):
        w = rows_v[t, pl.ds(i * 16, 16)]
        rows_v[t, pl.ds(i * 16, 16)] = plsc.bitcast(
            extract(w) * a + b, jnp.int32
        )
        return 0

      lax.fori_loop(0, chunk_vecs, norm_body, 0, unroll=8)

    def token_body(t):
      tid = plsc.load_gather(idx_v, [jnp.full((16,), c * CT + t, jnp.int32)])
      par = jnp.sum(tid & 1)

      @pl.when(par == 0)
      def _():
        half_body(t, lambda w: plsc.bitcast(w << 16, jnp.float32))

      @pl.when(par != 0)
      def _():
        half_body(t, lambda w: plsc.bitcast(w & mask, jnp.float32))

    for t in range(CT):
      token_body(t)

  start_in(0, r0, si0)
  start_in(1, r1, si1)

  def loop_body(k, _):
    for j in range(3):
      c = 3 * k + j
      b, si, so = bufs[j], isems[j], osems[j]
      nj = (j + 2) % 3
      nb, nsi, nso = bufs[nj], isems[nj], osems[nj]
      wait_in(b, si)
      compute(c, b)
      start_out(c, b, so)

      @pl.when(c + 2 < N_CHUNKS)
      def _():
        @pl.when(c >= 1)
        def _():
          wait_out(c - 1, nb, nso)

        start_in(c + 2, nb, nsi)

    return 0

  lax.fori_loop(0, (N_CHUNKS - 1) // 3, loop_body, 0)

  c_last = N_CHUNKS - 1
  wait_in(r0, si0)
  compute(c_last, r0)
  start_out(c_last, r0, so0)
  wait_out(c_last - 2, r1, so1)
  wait_out(c_last - 1, r2, so2)
  wait_out(c_last, r0, so0)


@jax.jit
def kernel(token_ids, PF_table):
  ids = token_ids.reshape(-1)
  mesh = plsc.VectorSubcoreMesh(core_axis_name="c", subcore_axis_name="s")
  fn = functools.partial(
      pl.kernel,
      out_type=jax.ShapeDtypeStruct((N_TOKENS, D), jnp.float32),
      mesh=mesh,
      compiler_params=pltpu.CompilerParams(needs_layout_passes=False),
      scratch_types=[
          pltpu.VMEM((N_PER_W,), jnp.int32),
          pltpu.VMEM((N_CHUNKS, CT), jnp.int32),
          pltpu.VMEM((CT, D), jnp.int32),
          pltpu.VMEM((CT, D), jnp.int32),
          pltpu.VMEM((CT, D), jnp.int32),
          pltpu.SemaphoreType.DMA,
          pltpu.SemaphoreType.DMA,
          pltpu.SemaphoreType.DMA,
          pltpu.SemaphoreType.DMA,
          pltpu.SemaphoreType.DMA,
          pltpu.SemaphoreType.DMA,
      ],
  )(_sc_body)
  out = fn(ids, PF_table)
  return out.reshape(*token_ids.shape, D)


# speedup vs baseline: 5.1157x; 1.0268x over previous
import functools

import jax
import jax.numpy as jnp
from jax import lax
from jax.experimental import pallas as pl
from jax.experimental.pallas import tpu as pltpu
from jax.experimental.pallas import tpu_sc as plsc

NC = 2
NS = 16
NW = NC * NS

CT = 4
D = 8192
N_TOKENS = 8192
N_PER_W = N_TOKENS // NW
N_CHUNKS = N_PER_W // CT


def _sc_body(idx_hbm, table_hbm, out_hbm, idx_v, widx2, r0, r1, r2,
             si0, si1, si2, so0, so1, so2):
  tbl32 = table_hbm.bitcast(jnp.int32)
  out32 = out_hbm.bitcast(jnp.int32)
  wid = lax.axis_index("s") * NC + lax.axis_index("c")
  base = wid * N_PER_W
  bufs = (r0, r1, r2)
  isems = (si0, si1, si2)
  osems = (so0, so1, so2)

  pltpu.sync_copy(idx_hbm.at[pl.ds(base, N_PER_W)], idx_v)
  lane = lax.iota(jnp.int32, 16)

  def widx_body(g, _):
    pos = g * 16 + lane
    v = idx_v[pl.ds(g * 16, 16)] >> 1
    plsc.store_scatter(widx2, [pos >> 2, pos & 3], v)
    return 0

  lax.fori_loop(0, N_PER_W // 16, widx_body, 0, unroll=4)

  chunk_vecs = D // 16
  inv_d = jnp.float32(1.0 / D)
  inv_dm1 = jnp.float32(1.0 / (D - 1))
  mask = jnp.int32(-65536)

  def start_in(c, b, s):
    pltpu.async_copy(tbl32.at[widx2.at[c]], b, s)

  def wait_in(b, s):
    pltpu.make_async_copy(tbl32.at[widx2.at[0]], b, s).wait()

  def start_out(c, b, s):
    pltpu.async_copy(b, out32.at[pl.ds(base + c * CT, CT)], s)

  def wait_out(c, b, s):
    pltpu.make_async_copy(b, out32.at[pl.ds(base + c * CT, CT)], s).wait()

  def compute(c, rows_v):
    def half_body(t, extract):
      def stats_body(i, carry):
        acc = list(carry)
        for u in range(4):
          w = rows_v[t, pl.ds((i * 4 + u) * 16, 16)]
          x = extract(w)
          acc[2 * u] = acc[2 * u] + x
          acc[2 * u + 1] = acc[2 * u + 1] + x * x
        return tuple(acc)

      z = jnp.zeros((16,), jnp.float32)
      acc = lax.fori_loop(0, chunk_vecs // 4, stats_body, (z,) * 8, unroll=4)
      tot = jnp.sum(acc[0] + acc[2] + acc[4] + acc[6])
      ss = jnp.sum(acc[1] + acc[3] + acc[5] + acc[7])
      mean = tot * inv_d
      var = (ss - tot * mean) * inv_dm1

      vv = jnp.full((16,), var, jnp.float32)
      iy = jnp.int32(0x5F3759DF) - (plsc.bitcast(vv, jnp.int32) >> 1)
      y = plsc.bitcast(iy, jnp.float32)
      half_v = vv * 0.5
      y = y * (1.5 - half_v * y * y)
      y = y * (1.5 - half_v * y * y)
      a = y - 1e-6 * (y * y)
      b = jnp.full((16,), -mean, jnp.float32) * a

      def norm_body(i, _):
        w = rows_v[t, pl.ds(i * 16, 16)]
        rows_v[t, pl.ds(i * 16, 16)] = plsc.bitcast(
            extract(w) * a + b, jnp.int32
        )
        return 0

      lax.fori_loop(0, chunk_vecs, norm_body, 0, unroll=16)

    def token_body(t):
      tid = plsc.load_gather(idx_v, [jnp.full((16,), c * CT + t, jnp.int32)])
      par = jnp.sum(tid & 1)

      @pl.when(par == 0)
      def _():
        half_body(t, lambda w: plsc.bitcast(w << 16, jnp.float32))

      @pl.when(par != 0)
      def _():
        half_body(t, lambda w: plsc.bitcast(w & mask, jnp.float32))

    for t in range(CT):
      token_body(t)

  start_in(0, r0, si0)
  start_in(1, r1, si1)

  def loop_body(k, _):
    for j in range(3):
      c = 3 * k + j
      b, si, so = bufs[j], isems[j], osems[j]
      nj = (j + 2) % 3
      nb, nsi, nso = bufs[nj], isems[nj], osems[nj]
      wait_in(b, si)
      compute(c, b)
      start_out(c, b, so)

      @pl.when(c + 2 < N_CHUNKS)
      def _():
        @pl.when(c >= 1)
        def _():
          wait_out(c - 1, nb, nso)

        start_in(c + 2, nb, nsi)

    return 0

  lax.fori_loop(0, (N_CHUNKS - 1) // 3, loop_body, 0)

  c_last = N_CHUNKS - 1
  wait_in(r0, si0)
  compute(c_last, r0)
  start_out(c_last, r0, so0)
  wait_out(c_last - 2, r1, so1)
  wait_out(c_last - 1, r2, so2)
  wait_out(c_last, r0, so0)


@jax.jit
def kernel(token_ids, PF_table):
  ids = token_ids.reshape(-1)
  mesh = plsc.VectorSubcoreMesh(core_axis_name="c", subcore_axis_name="s")
  fn = functools.partial(
      pl.kernel,
      out_type=jax.ShapeDtypeStruct((N_TOKENS, D), jnp.float32),
      mesh=mesh,
      compiler_params=pltpu.CompilerParams(needs_layout_passes=False),
      scratch_types=[
          pltpu.VMEM((N_PER_W,), jnp.int32),
          pltpu.VMEM((N_CHUNKS, CT), jnp.int32),
          pltpu.VMEM((CT, D), jnp.int32),
          pltpu.VMEM((CT, D), jnp.int32),
          pltpu.VMEM((CT, D), jnp.int32),
          pltpu.SemaphoreType.DMA,
          pltpu.SemaphoreType.DMA,
          pltpu.SemaphoreType.DMA,
          pltpu.SemaphoreType.DMA,
          pltpu.SemaphoreType.DMA,
          pltpu.SemaphoreType.DMA,
      ],
  )(_sc_body)
  out = fn(ids, PF_table)
  return out.reshape(*token_ids.shape, D)
